# baseline (speedup 1.0000x reference)
import functools

import jax
import jax.numpy as jnp
from jax import lax
from jax.experimental import pallas as pl
from jax.experimental.pallas import tpu as pltpu
from jax.experimental.pallas import tpu_sc as plsc

VOCAB = 1000000
DIM = 32
GROUP = 128
K = 5
NC = 2
NS = 16
NW = NC * NS
HIST = 200
BATCH = 4096
NJ = BATCH // GROUP
NI = HIST // 8
NT = -(-VOCAB // 128)
VPAD = NT * 128
TPW = -(-NT // NW)
TROWS = VPAD * DIM // 128


@functools.lru_cache(maxsize=None)
def _build_table():
    mesh = plsc.VectorSubcoreMesh(core_axis_name="c", subcore_axis_name="s")

    @functools.partial(
        pl.kernel,
        mesh=mesh,
        out_type=jax.ShapeDtypeStruct((TROWS, 128), jnp.float32),
        compiler_params=pltpu.CompilerParams(
            use_tc_tiling_on_sc=False, needs_layout_passes=False
        ),
        scratch_types=[
            pltpu.VMEM((2, 4, 8, 128), jnp.float32),
            pltpu.VMEM((2, DIM, 128), jnp.float32),
            pltpu.SemaphoreType.DMA((2,)),
            pltpu.SemaphoreType.DMA((2,)),
        ],
    )
    def k(ep4_hbm, tab_hbm, x_v, y_v, rsem, wsem):
        wid = lax.axis_index("s") * NC + lax.axis_index("c")
        iota = lax.iota(jnp.int32, 16)
        tmax = jnp.minimum((wid + 1) * TPW, NT)

        def fire_read(t, p):
            pltpu.make_async_copy(
                ep4_hbm.at[:, t], x_v.at[p], rsem.at[p]
            ).start()

        def transpose(p):
            def tbody(c, carry):
                for half in range(2):
                    d0 = 16 * half
                    l = c * DIM + d0
                    dv = d0 + iota
                    v = plsc.load_gather(
                        x_v.at[p],
                        [dv >> 3, dv & 7, jnp.full((16,), c, jnp.int32)],
                    )
                    y_v[p, l // 128, pl.ds(l % 128, 16)] = v
                return carry

            lax.fori_loop(0, 128, tbody, 0)

        def body(i, carry):
            for p in range(2):
                kl = 2 * i + p
                t = wid * TPW + kl

                @pl.when(t < tmax)
                def _():
                    pltpu.make_async_copy(
                        ep4_hbm.at[:, 0], x_v.at[p], rsem.at[p]
                    ).wait()

                    @pl.when(kl >= 2)
                    def _():
                        pltpu.make_async_copy(
                            y_v.at[p], tab_hbm.at[pl.ds(0, DIM)], wsem.at[p]
                        ).wait()

                    transpose(p)
                    pltpu.make_async_copy(
                        y_v.at[p],
                        tab_hbm.at[pl.ds(t * DIM, DIM)],
                        wsem.at[p],
                    ).start()

                    @pl.when(t + 2 < tmax)
                    def _():
                        fire_read(t + 2, p)

            return carry

        fire_read(wid * TPW, 0)

        @pl.when(wid * TPW + 1 < tmax)
        def _():
            fire_read(wid * TPW + 1, 1)

        lax.fori_loop(0, TPW // 2 + 1, body, 0)

        for p in range(2):
            @pl.when(wid * TPW + p < tmax)
            def _():
                pltpu.make_async_copy(
                    y_v.at[p], tab_hbm.at[pl.ds(0, DIM)], wsem.at[p]
                ).wait()

    return k


@functools.lru_cache(maxsize=None)
def _build_gather():
    chunks = HIST // K
    assert chunks % 2 == 0
    mesh = plsc.VectorSubcoreMesh(core_axis_name="c", subcore_axis_name="s")

    @functools.partial(
        pl.kernel,
        mesh=mesh,
        out_type=jax.ShapeDtypeStruct((HIST, 4, NJ, 8, 128), jnp.float32),
        compiler_params=pltpu.CompilerParams(
            use_tc_tiling_on_sc=False, needs_layout_passes=False
        ),
        scratch_types=[
            pltpu.VMEM((NI, 8, GROUP), jnp.int32),
            pltpu.VMEM((2, K, GROUP, DIM), jnp.float32),
            pltpu.VMEM((2, K, 4, 8, 128), jnp.float32),
            pltpu.SemaphoreType.DMA((2,)),
            pltpu.SemaphoreType.DMA((2,)),
        ],
    )
    def k(idx_hbm, table_hbm, out_hbm, idx_v, rows_v, y_v, gsem, wsem):
        wid = lax.axis_index("s") * NC + lax.axis_index("c")
        iota = lax.iota(jnp.int32, 16)
        pltpu.sync_copy(idx_hbm.at[:, wid], idx_v)

        def fire_chunk(c, p):
            for kk in range(K):
                h = c * K + kk
                pltpu.async_copy(
                    table_hbm.at[idx_v.at[h // 8, h % 8]],
                    rows_v.at[p, kk],
                    gsem.at[p],
                )

        def drain_gathers(p):
            for kk in range(K):
                pltpu.make_async_copy(
                    table_hbm.at[pl.ds(0, GROUP)], rows_v.at[p, kk], gsem.at[p]
                ).wait()

        def transpose_item(p, kk):
            def tbody(d, carry):
                for g in range(8):
                    c0 = g * 16
                    v = plsc.load_gather(
                        rows_v.at[p, kk],
                        [c0 + iota, jnp.full((16,), d, jnp.int32)],
                    )
                    y_v[p, kk, d // 8, d % 8, pl.ds(c0, 16)] = v
                return carry

            lax.fori_loop(0, DIM, tbody, 0)

        def drain_write(p):
            pltpu.make_async_copy(
                y_v.at[p], out_hbm.at[pl.ds(0, K), :, 0], wsem.at[p]
            ).wait()

        fire_chunk(0, 0)
        fire_chunk(1, 1)

        def body(i, carry):
            for p in range(2):
                c = 2 * i + p
                drain_gathers(p)

                @pl.when(i > 0)
                def _():
                    drain_write(p)

                for kk in range(K):
                    transpose_item(p, kk)

                pltpu.make_async_copy(
                    y_v.at[p], out_hbm.at[pl.ds(c * K, K), :, wid], wsem.at[p]
                ).start()

                @pl.when(i < chunks // 2 - 1)
                def _():
                    fire_chunk(c + 2, p)

            return carry

        lax.fori_loop(0, chunks // 2, body, 0)
        drain_write(0)
        drain_write(1)

    return k


def kernel(labels, E):
    ep = jnp.pad(E, ((0, VPAD - VOCAB), (0, 0)))
    ep4 = (
        ep.T.reshape(4, 8, NT, 128).transpose(0, 2, 1, 3)
    )
    elin = _build_table()(ep4)
    table = elin.reshape(VPAD, DIM)

    lab = labels.astype(jnp.int32)
    lab4 = lab.T.reshape(NI, 8, NJ, GROUP).transpose(0, 2, 1, 3)

    out5 = _build_gather()(lab4, table)
    return out5.transpose(2, 4, 0, 1, 3).reshape(BATCH, HIST, DIM)


# speedup vs baseline: 1.2172x; 1.2172x over previous
import functools

import jax
import jax.numpy as jnp
from jax import lax
from jax.experimental import pallas as pl
from jax.experimental.pallas import tpu as pltpu
from jax.experimental.pallas import tpu_sc as plsc

VOCAB = 1000000
DIM = 32
GROUP = 128
K = 5
NC = 2
NS = 16
NW = NC * NS
HIST = 200
BATCH = 4096
NJ = BATCH // GROUP
NI = HIST // 8
NT = -(-VOCAB // 128)
VPAD = NT * 128
TPW = -(-NT // NW)
TROWS = VPAD * DIM // 128


@functools.lru_cache(maxsize=None)
def _build_table():
    mesh = plsc.VectorSubcoreMesh(core_axis_name="c", subcore_axis_name="s")

    @functools.partial(
        pl.kernel,
        mesh=mesh,
        out_type=jax.ShapeDtypeStruct((TROWS, 128), jnp.float32),
        compiler_params=pltpu.CompilerParams(
            use_tc_tiling_on_sc=False, needs_layout_passes=False
        ),
        scratch_types=[
            pltpu.VMEM((2, 4, 8, 128), jnp.float32),
            pltpu.VMEM((2, DIM, 128), jnp.float32),
            pltpu.SemaphoreType.DMA((2,)),
            pltpu.SemaphoreType.DMA((2,)),
        ],
    )
    def k(ep4_hbm, tab_hbm, x_v, y_v, rsem, wsem):
        wid = lax.axis_index("s") * NC + lax.axis_index("c")
        iota = lax.iota(jnp.int32, 16)
        tmax = jnp.minimum((wid + 1) * TPW, NT)

        def fire_read(t, p):
            pltpu.make_async_copy(
                ep4_hbm.at[:, t], x_v.at[p], rsem.at[p]
            ).start()

        cg32 = [(16 * g + iota) * DIM for g in range(8)]

        def transpose(p):
            def tbody(d, carry):
                pp = d >> 3
                q = d & 7
                for g in range(8):
                    v = x_v[p, pp, q, pl.ds(16 * g, 16)]
                    lv = cg32[g] + d
                    plsc.store_scatter(y_v.at[p], [lv >> 7, lv & 127], v)
                return carry

            lax.fori_loop(0, DIM, tbody, 0)

        def body(i, carry):
            for p in range(2):
                kl = 2 * i + p
                t = wid * TPW + kl

                @pl.when(t < tmax)
                def _():
                    pltpu.make_async_copy(
                        ep4_hbm.at[:, 0], x_v.at[p], rsem.at[p]
                    ).wait()

                    @pl.when(kl >= 2)
                    def _():
                        pltpu.make_async_copy(
                            y_v.at[p], tab_hbm.at[pl.ds(0, DIM)], wsem.at[p]
                        ).wait()

                    transpose(p)
                    pltpu.make_async_copy(
                        y_v.at[p],
                        tab_hbm.at[pl.ds(t * DIM, DIM)],
                        wsem.at[p],
                    ).start()

                    @pl.when(t + 2 < tmax)
                    def _():
                        fire_read(t + 2, p)

            return carry

        fire_read(wid * TPW, 0)

        @pl.when(wid * TPW + 1 < tmax)
        def _():
            fire_read(wid * TPW + 1, 1)

        lax.fori_loop(0, TPW // 2 + 1, body, 0)

        for p in range(2):
            @pl.when(wid * TPW + p < tmax)
            def _():
                pltpu.make_async_copy(
                    y_v.at[p], tab_hbm.at[pl.ds(0, DIM)], wsem.at[p]
                ).wait()

    return k


@functools.lru_cache(maxsize=None)
def _build_gather():
    chunks = HIST // K
    assert chunks % 2 == 0
    mesh = plsc.VectorSubcoreMesh(core_axis_name="c", subcore_axis_name="s")

    @functools.partial(
        pl.kernel,
        mesh=mesh,
        out_type=jax.ShapeDtypeStruct((HIST, 4, NJ, 8, 128), jnp.float32),
        compiler_params=pltpu.CompilerParams(
            use_tc_tiling_on_sc=False, needs_layout_passes=False
        ),
        scratch_types=[
            pltpu.VMEM((NI, 8, GROUP), jnp.int32),
            pltpu.VMEM((2, K, GROUP, DIM), jnp.float32),
            pltpu.VMEM((2, K, 4, 8, 128), jnp.float32),
            pltpu.SemaphoreType.DMA((2,)),
            pltpu.SemaphoreType.DMA((2,)),
        ],
    )
    def k(idx_hbm, table_hbm, out_hbm, idx_v, rows_v, y_v, gsem, wsem):
        wid = lax.axis_index("s") * NC + lax.axis_index("c")
        iota = lax.iota(jnp.int32, 16)
        pltpu.sync_copy(idx_hbm.at[:, wid], idx_v)

        def fire_chunk(c, p):
            for kk in range(K):
                h = c * K + kk
                pltpu.async_copy(
                    table_hbm.at[idx_v.at[h // 8, h % 8]],
                    rows_v.at[p, kk],
                    gsem.at[p],
                )

        def drain_gathers(p):
            for kk in range(K):
                pltpu.make_async_copy(
                    table_hbm.at[pl.ds(0, GROUP)], rows_v.at[p, kk], gsem.at[p]
                ).wait()

        dv0 = iota >> 3
        dq0 = iota & 7
        dv1 = (16 + iota) >> 3
        dq1 = (16 + iota) & 7

        def transpose_item(p, kk):
            def tbody(c, carry):
                cv = jnp.full((16,), c, jnp.int32)
                v0 = rows_v[p, kk, c, pl.ds(0, 16)]
                plsc.store_scatter(y_v.at[p, kk], [dv0, dq0, cv], v0)
                v1 = rows_v[p, kk, c, pl.ds(16, 16)]
                plsc.store_scatter(y_v.at[p, kk], [dv1, dq1, cv], v1)
                return carry

            lax.fori_loop(0, GROUP, tbody, 0)

        def drain_write(p):
            pltpu.make_async_copy(
                y_v.at[p], out_hbm.at[pl.ds(0, K), :, 0], wsem.at[p]
            ).wait()

        fire_chunk(0, 0)
        fire_chunk(1, 1)

        def body(i, carry):
            for p in range(2):
                c = 2 * i + p
                drain_gathers(p)

                @pl.when(i > 0)
                def _():
                    drain_write(p)

                for kk in range(K):
                    transpose_item(p, kk)

                pltpu.make_async_copy(
                    y_v.at[p], out_hbm.at[pl.ds(c * K, K), :, wid], wsem.at[p]
                ).start()

                @pl.when(i < chunks // 2 - 1)
                def _():
                    fire_chunk(c + 2, p)

            return carry

        lax.fori_loop(0, chunks // 2, body, 0)
        drain_write(0)
        drain_write(1)

    return k


def kernel(labels, E):
    ep = jnp.pad(E, ((0, VPAD - VOCAB), (0, 0)))
    ep4 = (
        ep.T.reshape(4, 8, NT, 128).transpose(0, 2, 1, 3)
    )
    elin = _build_table()(ep4)
    table = elin.reshape(VPAD, DIM)

    lab = labels.astype(jnp.int32)
    lab4 = lab.T.reshape(NI, 8, NJ, GROUP).transpose(0, 2, 1, 3)

    out5 = _build_gather()(lab4, table)
    return out5.transpose(2, 4, 0, 1, 3).reshape(BATCH, HIST, DIM)


# speedup vs baseline: 2.4882x; 2.0442x over previous
import functools

import jax
import jax.numpy as jnp
from jax import lax
from jax.experimental import pallas as pl
from jax.experimental.pallas import tpu as pltpu
from jax.experimental.pallas import tpu_sc as plsc

VOCAB = 1000000
DIM = 32
GROUP = 128
K = 5
NC = 2
NS = 16
NW = NC * NS
HIST = 200
BATCH = 4096
NJ = BATCH // GROUP
NI = HIST // 8
NT = -(-VOCAB // 128)
VPAD = NT * 128
TPW = -(-NT // NW)
TROWS = VPAD * DIM // 128


@functools.lru_cache(maxsize=None)
def _build_table():
    mesh = plsc.VectorSubcoreMesh(core_axis_name="c", subcore_axis_name="s")

    @functools.partial(
        pl.kernel,
        mesh=mesh,
        out_type=jax.ShapeDtypeStruct((TROWS, 128), jnp.float32),
        compiler_params=pltpu.CompilerParams(
            use_tc_tiling_on_sc=False, needs_layout_passes=False
        ),
        scratch_types=[
            pltpu.VMEM((2, 4, 8, 128), jnp.float32),
            pltpu.VMEM((2, DIM, 128), jnp.float32),
            pltpu.SemaphoreType.DMA((2,)),
            pltpu.SemaphoreType.DMA((2,)),
        ],
    )
    def k(ep4_hbm, tab_hbm, x_v, y_v, rsem, wsem):
        wid = lax.axis_index("s") * NC + lax.axis_index("c")
        iota = lax.iota(jnp.int32, 16)
        tmax = jnp.minimum((wid + 1) * TPW, NT)

        def fire_read(t, p):
            pltpu.make_async_copy(
                ep4_hbm.at[:, t], x_v.at[p], rsem.at[p]
            ).start()

        def transpose(p):
            def tbody(cb, carry):
                cvec = cb * 16 + iota
                base32 = cvec * DIM
                for d0 in (0, 16):
                    for kk in range(16):
                        dd = d0 + ((iota + kk) & 15)
                        v = plsc.load_gather(
                            x_v.at[p], [dd >> 3, dd & 7, cvec]
                        )
                        lv = base32 + dd
                        plsc.store_scatter(
                            y_v.at[p], [lv >> 7, lv & 127], v
                        )
                return carry

            lax.fori_loop(0, 8, tbody, 0)

        def body(i, carry):
            for p in range(2):
                kl = 2 * i + p
                t = wid * TPW + kl

                @pl.when(t < tmax)
                def _():
                    pltpu.make_async_copy(
                        ep4_hbm.at[:, 0], x_v.at[p], rsem.at[p]
                    ).wait()

                    @pl.when(kl >= 2)
                    def _():
                        pltpu.make_async_copy(
                            y_v.at[p], tab_hbm.at[pl.ds(0, DIM)], wsem.at[p]
                        ).wait()

                    transpose(p)
                    pltpu.make_async_copy(
                        y_v.at[p],
                        tab_hbm.at[pl.ds(t * DIM, DIM)],
                        wsem.at[p],
                    ).start()

                    @pl.when(t + 2 < tmax)
                    def _():
                        fire_read(t + 2, p)

            return carry

        fire_read(wid * TPW, 0)

        @pl.when(wid * TPW + 1 < tmax)
        def _():
            fire_read(wid * TPW + 1, 1)

        lax.fori_loop(0, TPW // 2 + 1, body, 0)

        for p in range(2):
            @pl.when(wid * TPW + p < tmax)
            def _():
                pltpu.make_async_copy(
                    y_v.at[p], tab_hbm.at[pl.ds(0, DIM)], wsem.at[p]
                ).wait()

    return k


@functools.lru_cache(maxsize=None)
def _build_gather():
    chunks = HIST // K
    assert chunks % 2 == 0
    mesh = plsc.VectorSubcoreMesh(core_axis_name="c", subcore_axis_name="s")

    @functools.partial(
        pl.kernel,
        mesh=mesh,
        out_type=jax.ShapeDtypeStruct((HIST, 4, NJ, 8, 128), jnp.float32),
        compiler_params=pltpu.CompilerParams(
            use_tc_tiling_on_sc=False, needs_layout_passes=False
        ),
        scratch_types=[
            pltpu.VMEM((NI, 8, GROUP), jnp.int32),
            pltpu.VMEM((2, K, GROUP, DIM), jnp.float32),
            pltpu.VMEM((2, K, 4, 8, 128), jnp.float32),
            pltpu.SemaphoreType.DMA((2,)),
            pltpu.SemaphoreType.DMA((2,)),
        ],
    )
    def k(idx_hbm, table_hbm, out_hbm, idx_v, rows_v, y_v, gsem, wsem):
        wid = lax.axis_index("s") * NC + lax.axis_index("c")
        iota = lax.iota(jnp.int32, 16)
        pltpu.sync_copy(idx_hbm.at[:, wid], idx_v)

        def fire_chunk(c, p):
            for kk in range(K):
                h = c * K + kk
                pltpu.async_copy(
                    table_hbm.at[idx_v.at[h // 8, h % 8]],
                    rows_v.at[p, kk],
                    gsem.at[p],
                )

        def drain_gathers(p):
            for kk in range(K):
                pltpu.make_async_copy(
                    table_hbm.at[pl.ds(0, GROUP)], rows_v.at[p, kk], gsem.at[p]
                ).wait()

        def transpose_item(p, kk):
            def tbody(cb, carry):
                cvec = cb * 16 + iota
                for d0 in (0, 16):
                    for kr in range(16):
                        dd = d0 + ((iota + kr) & 15)
                        v = plsc.load_gather(rows_v.at[p, kk], [cvec, dd])
                        plsc.store_scatter(
                            y_v.at[p, kk], [dd >> 3, dd & 7, cvec], v
                        )
                return carry

            lax.fori_loop(0, 8, tbody, 0)

        def drain_write(p):
            pltpu.make_async_copy(
                y_v.at[p], out_hbm.at[pl.ds(0, K), :, 0], wsem.at[p]
            ).wait()

        fire_chunk(0, 0)
        fire_chunk(1, 1)

        def body(i, carry):
            for p in range(2):
                c = 2 * i + p
                drain_gathers(p)

                @pl.when(i > 0)
                def _():
                    drain_write(p)

                for kk in range(K):
                    transpose_item(p, kk)

                pltpu.make_async_copy(
                    y_v.at[p], out_hbm.at[pl.ds(c * K, K), :, wid], wsem.at[p]
                ).start()

                @pl.when(i < chunks // 2 - 1)
                def _():
                    fire_chunk(c + 2, p)

            return carry

        lax.fori_loop(0, chunks // 2, body, 0)
        drain_write(0)
        drain_write(1)

    return k


def kernel(labels, E):
    ep = jnp.pad(E, ((0, VPAD - VOCAB), (0, 0)))
    ep4 = (
        ep.T.reshape(4, 8, NT, 128).transpose(0, 2, 1, 3)
    )
    elin = _build_table()(ep4)
    table = elin.reshape(VPAD, DIM)

    lab = labels.astype(jnp.int32)
    lab4 = lab.T.reshape(NI, 8, NJ, GROUP).transpose(0, 2, 1, 3)

    out5 = _build_gather()(lab4, table)
    return out5.transpose(2, 4, 0, 1, 3).reshape(BATCH, HIST, DIM)


# speedup vs baseline: 2.8751x; 1.1555x over previous
import functools

import jax
import jax.numpy as jnp
from jax import lax
from jax.experimental import pallas as pl
from jax.experimental.pallas import tpu as pltpu
from jax.experimental.pallas import tpu_sc as plsc

VOCAB = 1000000
DIM = 32
GROUP = 128
K = 5
NC = 2
NS = 16
NW = NC * NS
HIST = 200
BATCH = 4096
NJ = BATCH // GROUP
NI = HIST // 8
NT = -(-VOCAB // 128)
VPAD = NT * 128
TPW = -(-NT // NW)
TROWS = VPAD * DIM // 128


@functools.lru_cache(maxsize=None)
def _build_table():
    mesh = plsc.VectorSubcoreMesh(core_axis_name="c", subcore_axis_name="s")

    @functools.partial(
        pl.kernel,
        mesh=mesh,
        out_type=jax.ShapeDtypeStruct((TROWS, 128), jnp.float32),
        compiler_params=pltpu.CompilerParams(needs_layout_passes=False),
        scratch_types=[
            pltpu.VMEM((2, DIM, 128), jnp.float32),
            pltpu.VMEM((2, DIM, 128), jnp.float32),
            pltpu.VMEM((DIM, 64), jnp.float32),
            pltpu.SemaphoreType.DMA((2,)),
            pltpu.SemaphoreType.DMA((2,)),
        ],
    )
    def k(et_hbm, tab_hbm, x_v, y_v, xt_v, rsem, wsem):
        wid = lax.axis_index("s") * NC + lax.axis_index("c")
        iota = lax.iota(jnp.int32, 16)
        tmax = jnp.minimum((wid + 1) * TPW, NT - 1)

        def fire_read(t, p):
            pltpu.make_async_copy(
                et_hbm.at[:, pl.ds(t * 128, 128)], x_v.at[p], rsem.at[p]
            ).start()

        def transpose(p):
            def tbody(cb, carry):
                cvec = cb * 16 + iota
                base32 = cvec * DIM
                for d0 in (0, 16):
                    for kk in range(16):
                        dd = d0 + ((iota + kk) & 15)
                        v = plsc.load_gather(x_v.at[p], [dd, cvec])
                        lv = base32 + dd
                        plsc.store_scatter(
                            y_v.at[p], [lv >> 7, lv & 127], v
                        )
                return carry

            lax.fori_loop(0, 8, tbody, 0)

        def body(i, carry):
            for p in range(2):
                kl = 2 * i + p
                t = wid * TPW + kl

                @pl.when(t < tmax)
                def _():
                    pltpu.make_async_copy(
                        et_hbm.at[:, pl.ds(0, 128)], x_v.at[p], rsem.at[p]
                    ).wait()

                    @pl.when(kl >= 2)
                    def _():
                        pltpu.make_async_copy(
                            y_v.at[p], tab_hbm.at[pl.ds(0, DIM)], wsem.at[p]
                        ).wait()

                    transpose(p)
                    pltpu.make_async_copy(
                        y_v.at[p],
                        tab_hbm.at[pl.ds(t * DIM, DIM)],
                        wsem.at[p],
                    ).start()

                    @pl.when(t + 2 < tmax)
                    def _():
                        fire_read(t + 2, p)

            return carry

        fire_read(wid * TPW, 0)

        @pl.when(wid * TPW + 1 < tmax)
        def _():
            fire_read(wid * TPW + 1, 1)

        lax.fori_loop(0, TPW // 2 + 1, body, 0)

        for p in range(2):
            @pl.when(wid * TPW + p < tmax)
            def _():
                pltpu.make_async_copy(
                    y_v.at[p], tab_hbm.at[pl.ds(0, DIM)], wsem.at[p]
                ).wait()

        @pl.when(wid == NW - 1)
        def _():
            pltpu.sync_copy(et_hbm.at[:, pl.ds((NT - 1) * 128, 64)], xt_v)

            def tbody(cb, carry):
                cvec = cb * 16 + iota
                base32 = cvec * DIM
                for d0 in (0, 16):
                    for kk in range(16):
                        dd = d0 + ((iota + kk) & 15)
                        v = plsc.load_gather(xt_v, [dd, cvec])
                        lv = base32 + dd
                        plsc.store_scatter(
                            y_v.at[0], [lv >> 7, lv & 127], v
                        )
                return carry

            lax.fori_loop(0, 4, tbody, 0)
            pltpu.sync_copy(
                y_v.at[0, pl.ds(0, 16)],
                tab_hbm.at[pl.ds((NT - 1) * DIM, 16)],
            )

    return k


@functools.lru_cache(maxsize=None)
def _build_gather():
    chunks = HIST // K
    assert chunks % 2 == 0
    mesh = plsc.VectorSubcoreMesh(core_axis_name="c", subcore_axis_name="s")

    @functools.partial(
        pl.kernel,
        mesh=mesh,
        out_type=jax.ShapeDtypeStruct((HIST, 4, NJ, 8, 128), jnp.float32),
        compiler_params=pltpu.CompilerParams(
            use_tc_tiling_on_sc=False, needs_layout_passes=False
        ),
        scratch_types=[
            pltpu.VMEM((NI, 8, GROUP), jnp.int32),
            pltpu.VMEM((2, K, GROUP, DIM), jnp.float32),
            pltpu.VMEM((2, K, 4, 8, 128), jnp.float32),
            pltpu.SemaphoreType.DMA((2,)),
            pltpu.SemaphoreType.DMA((2,)),
        ],
    )
    def k(idx_hbm, table_hbm, out_hbm, idx_v, rows_v, y_v, gsem, wsem):
        wid = lax.axis_index("s") * NC + lax.axis_index("c")
        iota = lax.iota(jnp.int32, 16)
        pltpu.sync_copy(idx_hbm.at[:, wid], idx_v)

        def fire_chunk(c, p):
            for kk in range(K):
                h = c * K + kk
                pltpu.async_copy(
                    table_hbm.at[idx_v.at[h // 8, h % 8]],
                    rows_v.at[p, kk],
                    gsem.at[p],
                )

        def drain_gathers(p):
            for kk in range(K):
                pltpu.make_async_copy(
                    table_hbm.at[pl.ds(0, GROUP)], rows_v.at[p, kk], gsem.at[p]
                ).wait()

        def transpose_item(p, kk):
            def tbody(cb, carry):
                cvec = cb * 16 + iota
                for d0 in (0, 16):
                    for kr in range(16):
                        dd = d0 + ((iota + kr) & 15)
                        v = plsc.load_gather(rows_v.at[p, kk], [cvec, dd])
                        plsc.store_scatter(
                            y_v.at[p, kk], [dd >> 3, dd & 7, cvec], v
                        )
                return carry

            lax.fori_loop(0, 8, tbody, 0)

        def drain_write(p):
            pltpu.make_async_copy(
                y_v.at[p], out_hbm.at[pl.ds(0, K), :, 0], wsem.at[p]
            ).wait()

        fire_chunk(0, 0)
        fire_chunk(1, 1)

        def body(i, carry):
            for p in range(2):
                c = 2 * i + p
                drain_gathers(p)

                @pl.when(i > 0)
                def _():
                    drain_write(p)

                for kk in range(K):
                    transpose_item(p, kk)

                pltpu.make_async_copy(
                    y_v.at[p], out_hbm.at[pl.ds(c * K, K), :, wid], wsem.at[p]
                ).start()

                @pl.when(i < chunks // 2 - 1)
                def _():
                    fire_chunk(c + 2, p)

            return carry

        lax.fori_loop(0, chunks // 2, body, 0)
        drain_write(0)
        drain_write(1)

    return k


def kernel(labels, E):
    et = E.T
    elin = _build_table()(et)
    table = elin.reshape(VPAD, DIM)

    lab = labels.astype(jnp.int32)
    lab4 = lab.T.reshape(NI, 8, NJ, GROUP).transpose(0, 2, 1, 3)

    out5 = _build_gather()(lab4, table)
    return out5.transpose(2, 4, 0, 1, 3).reshape(BATCH, HIST, DIM)
